# baseline (speedup 1.0000x reference)
import jax
import jax.numpy as jnp
from jax import lax
from jax.experimental import pallas as pl
from jax.experimental.pallas import tpu as pltpu
from jax.experimental.pallas import tpu_sc as plsc

_BATCH = 16384
_NC = 2
_NS = 16
_NW = _NC * _NS
_BPW = _BATCH // _NW
_CHUNK = 128
_CHUNKS = _BPW // _CHUNK
_GROUPS = _BPW // 16
_TILES = 782
_TW = 8


def _sc_body(idx_hbm, table_hbm, out_hbm, idx_v, klist, evals, rows_v, out_v,
             sems):
    wid = lax.axis_index("s") * _NC + lax.axis_index("c")
    pltpu.sync_copy(idx_hbm.at[wid], idx_v)

    lanes = lax.iota(jnp.int32, 16)

    def _chunk_copies(j):
        return [
            pltpu.make_async_copy(
                table_hbm.at[klist.at[c * _CHUNKS + j]],
                rows_v.at[pl.ds((c * _CHUNKS + j) * _CHUNK, _CHUNK)],
                sems,
            )
            for c in range(6)
        ]

    for j in range(_CHUNKS):
        def rowcalc(q, carry, j=j):
            i = idx_v[j, pl.ds(q * 16, 16)]
            base = ((i >> 7) << 6) | ((i >> 4) & 7)
            evals[j, pl.ds(q * 16, 16)] = i & 15
            for c in range(6):
                klist[c * _CHUNKS + j, pl.ds(q * 16, 16)] = base | (c << 3)
            return carry

        lax.fori_loop(0, 8, rowcalc, 0)
        for d in _chunk_copies(j):
            d.start()

    def group(g, carry):
        j = g >> 3
        sub = (g & 7) * 16
        e = evals[j, pl.ds(sub, 16)]

        def col(c):
            row = (c * _CHUNKS + j) * _CHUNK + sub + lanes
            return plsc.load_gather(rows_v, [row, e])

        t0, t1, t2 = col(0), col(1), col(2)
        w0, w1, w2 = col(3), col(4), col(5)
        n = jnp.maximum(w0 * w0 + w1 * w1 + w2 * w2, 1e-4)
        f1 = 1.0 + n * (-1.0 / 6.0 + n * (1.0 / 120.0 + n * (-1.0 / 5040.0 + n * (1.0 / 362880.0))))
        f2 = 0.5 + n * (-1.0 / 24.0 + n * (1.0 / 720.0 + n * (-1.0 / 40320.0 + n * (1.0 / 3628800.0))))
        a0, a1, a2 = f1 * w0, f1 * w1, f1 * w2
        b01, b02, b12 = f2 * w0 * w1, f2 * w0 * w2, f2 * w1 * w2
        d0 = 1.0 + f2 * (w0 * w0 - n)
        d1 = 1.0 + f2 * (w1 * w1 - n)
        d2 = 1.0 + f2 * (w2 * w2 - n)
        vals = (d0, b01 - a2, b02 + a1, t0,
                b01 + a2, d1, b12 - a0, t1,
                b02 - a1, b12 + a0, d2, t2)
        base = (g >> 3) * 512 + (g & 7) * 16
        for k, v in enumerate(vals):
            r, c2 = k // 4, k % 4
            out_v[r, pl.ds(c2 * 128 + base, 16)] = v
        return carry

    for j in range(_CHUNKS):
        for d in _chunk_copies(j):
            d.wait()
    lax.fori_loop(0, _GROUPS, group, 0)
    pltpu.sync_copy(out_v, out_hbm.at[:, pl.ds(wid * 16 * _CHUNK, 16 * _CHUNK)])


_sc_kernel = pl.kernel(
    _sc_body,
    out_type=jax.ShapeDtypeStruct((3, _BATCH * 4), jnp.float32),
    mesh=plsc.VectorSubcoreMesh(core_axis_name="c", subcore_axis_name="s"),
    compiler_params=pltpu.CompilerParams(
        needs_layout_passes=False, use_tc_tiling_on_sc=False),
    scratch_types=[
        pltpu.VMEM((_CHUNKS, _CHUNK), jnp.int32),
        pltpu.VMEM((6 * _CHUNKS, _CHUNK), jnp.int32),
        pltpu.VMEM((_CHUNKS, _CHUNK), jnp.int32),
        pltpu.VMEM((6 * _CHUNKS * _CHUNK, 2 * _TW), jnp.float32),
        pltpu.VMEM((3, 16 * _CHUNK), jnp.float32),
        pltpu.SemaphoreType.DMA,
    ],
)


@jax.jit
def kernel(indices, pose_adjustment):
    idx = indices.astype(jnp.int32).reshape(_NW, _CHUNKS, _CHUNK)
    traw = jnp.pad(pose_adjustment.T, ((0, 2), (0, 96)))
    table = traw.reshape(_TW, _TILES, _CHUNK).transpose(1, 0, 2)
    table = table.reshape(_TILES * 64, 2 * _TW)
    out = _sc_kernel(idx, table)
    out4 = out.reshape(3, _BATCH // _CHUNK, 4, _CHUNK)
    return out4.transpose(1, 3, 0, 2).reshape(_BATCH, 3, 4)


# speedup vs baseline: 1.0075x; 1.0075x over previous
import jax
import jax.numpy as jnp
from jax import lax
from jax.experimental import pallas as pl
from jax.experimental.pallas import tpu as pltpu
from jax.experimental.pallas import tpu_sc as plsc

_BATCH = 16384
_NC = 2
_NS = 16
_NW = _NC * _NS
_BPW = _BATCH // _NW
_CHUNK = 128
_CHUNKS = _BPW // _CHUNK
_GROUPS = _BPW // 16
_TILES = 782
_TW = 8


def _sc_body(idx_hbm, table_hbm, out_hbm, idx_v, klist, evals, rows_v, out_v, sem):
    wid = lax.axis_index("s") * _NC + lax.axis_index("c")
    pltpu.sync_copy(idx_hbm.at[wid], idx_v)

    lanes = lax.iota(jnp.int32, 16)

    def rowcalc(m, carry):
        j, q = m >> 3, m & 7
        i = idx_v[j, pl.ds(q * 16, 16)]
        base = ((i >> 7) << 7) | ((i >> 3) & 15)
        evals[j, pl.ds(q * 16, 16)] = i & 7
        for c in range(6):
            klist[c * _CHUNKS + j, pl.ds(q * 16, 16)] = base | (c << 4)
        return carry

    lax.fori_loop(0, 8 * _CHUNKS, rowcalc, 0)

    descs = []
    for c in range(6):
        for j in range(_CHUNKS):
            d = pltpu.make_async_copy(
                table_hbm.at[klist.at[c * _CHUNKS + j]],
                rows_v.at[pl.ds((c * _CHUNKS + j) * _CHUNK, _CHUNK)],
                sem,
            )
            d.start()
            descs.append(d)
    for d in descs:
        d.wait()

    def group(g, carry):
        j = g >> 3
        sub = (g & 7) * 16
        e = evals[j, pl.ds(sub, 16)]

        def col(c):
            row = (c * _CHUNKS + j) * _CHUNK + sub + lanes
            return plsc.load_gather(rows_v, [row, e])

        t0, t1, t2 = col(0), col(1), col(2)
        w0, w1, w2 = col(3), col(4), col(5)
        n = jnp.maximum(w0 * w0 + w1 * w1 + w2 * w2, 1e-4)
        f1 = 1.0 + n * (-1.0 / 6.0 + n * (1.0 / 120.0 + n * (-1.0 / 5040.0 + n * (1.0 / 362880.0))))
        f2 = 0.5 + n * (-1.0 / 24.0 + n * (1.0 / 720.0 + n * (-1.0 / 40320.0 + n * (1.0 / 3628800.0))))
        a0, a1, a2 = f1 * w0, f1 * w1, f1 * w2
        b01, b02, b12 = f2 * w0 * w1, f2 * w0 * w2, f2 * w1 * w2
        d0 = 1.0 + f2 * (w0 * w0 - n)
        d1 = 1.0 + f2 * (w1 * w1 - n)
        d2 = 1.0 + f2 * (w2 * w2 - n)
        vals = (d0, b01 - a2, b02 + a1, t0,
                b01 + a2, d1, b12 - a0, t1,
                b02 - a1, b12 + a0, d2, t2)
        base = (g >> 3) * 512 + (g & 7) * 16
        for k, v in enumerate(vals):
            r, c2 = k // 4, k % 4
            out_v[r, pl.ds(c2 * 128 + base, 16)] = v
        return carry

    lax.fori_loop(0, _GROUPS, group, 0)
    pltpu.sync_copy(out_v, out_hbm.at[:, pl.ds(wid * 16 * _CHUNK, 16 * _CHUNK)])


_sc_kernel = pl.kernel(
    _sc_body,
    out_type=jax.ShapeDtypeStruct((3, _BATCH * 4), jnp.float32),
    mesh=plsc.VectorSubcoreMesh(core_axis_name="c", subcore_axis_name="s"),
    compiler_params=pltpu.CompilerParams(
        needs_layout_passes=False, use_tc_tiling_on_sc=False),
    scratch_types=[
        pltpu.VMEM((_CHUNKS, _CHUNK), jnp.int32),
        pltpu.VMEM((6 * _CHUNKS, _CHUNK), jnp.int32),
        pltpu.VMEM((_CHUNKS, _CHUNK), jnp.int32),
        pltpu.VMEM((6 * _CHUNKS * _CHUNK, _TW), jnp.float32),
        pltpu.VMEM((3, 16 * _CHUNK), jnp.float32),
        pltpu.SemaphoreType.DMA,
    ],
)


@jax.jit
def kernel(indices, pose_adjustment):
    idx = indices.astype(jnp.int32).reshape(_NW, _CHUNKS, _CHUNK)
    traw = jnp.pad(pose_adjustment.T, ((0, 2), (0, 96)))
    table = traw.reshape(_TW, _TILES, _CHUNK).transpose(1, 0, 2)
    table = table.reshape(_TILES * _CHUNK, _TW)
    out = _sc_kernel(idx, table)
    out4 = out.reshape(3, _BATCH // _CHUNK, 4, _CHUNK)
    return out4.transpose(1, 3, 0, 2).reshape(_BATCH, 3, 4)


# speedup vs baseline: 1.0087x; 1.0012x over previous
import jax
import jax.numpy as jnp
from jax import lax
from jax.experimental import pallas as pl
from jax.experimental.pallas import tpu as pltpu
from jax.experimental.pallas import tpu_sc as plsc

_BATCH = 16384
_NC = 2
_NS = 16
_NW = _NC * _NS
_BPW = _BATCH // _NW
_CHUNK = 128
_CHUNKS = _BPW // _CHUNK
_GROUPS = _BPW // 16
_TILES = 782
_TW = 8


def _sc_body(idx_hbm, table_hbm, out_hbm, idx_v, klist, rows_v, out_v, sem):
    wid = lax.axis_index("s") * _NC + lax.axis_index("c")
    pltpu.sync_copy(idx_hbm.at[wid], idx_v)

    lanes = lax.iota(jnp.int32, 16)

    def rowcalc(m, carry):
        j, q = m >> 3, m & 7
        i = idx_v[j, pl.ds(q * 16, 16)]
        klist[j, pl.ds(q * 16, 16)] = ((i >> 7) << 7) | ((i >> 3) & 15)
        return carry

    lax.fori_loop(0, 8 * _CHUNKS, rowcalc, 0)

    descs = []
    for c in range(6):
        view = table_hbm.at[pl.ds(c * 16, (_TILES - 1) * _CHUNK + 16)]
        for j in range(_CHUNKS):
            d = pltpu.make_async_copy(
                view.at[klist.at[j]],
                rows_v.at[pl.ds((c * _CHUNKS + j) * _CHUNK, _CHUNK)],
                sem,
            )
            d.start()
            descs.append(d)
    for d in descs:
        d.wait()

    def group(g, carry):
        j = g >> 3
        sub = (g & 7) * 16
        e = idx_v[j, pl.ds(sub, 16)] & 7

        def col(c):
            row = (c * _CHUNKS + j) * _CHUNK + sub + lanes
            return plsc.load_gather(rows_v, [row, e])

        t0, t1, t2 = col(0), col(1), col(2)
        w0, w1, w2 = col(3), col(4), col(5)
        n = jnp.maximum(w0 * w0 + w1 * w1 + w2 * w2, 1e-4)
        f1 = 1.0 + n * (-1.0 / 6.0 + n * (1.0 / 120.0 + n * (-1.0 / 5040.0 + n * (1.0 / 362880.0))))
        f2 = 0.5 + n * (-1.0 / 24.0 + n * (1.0 / 720.0 + n * (-1.0 / 40320.0 + n * (1.0 / 3628800.0))))
        a0, a1, a2 = f1 * w0, f1 * w1, f1 * w2
        b01, b02, b12 = f2 * w0 * w1, f2 * w0 * w2, f2 * w1 * w2
        d0 = 1.0 + f2 * (w0 * w0 - n)
        d1 = 1.0 + f2 * (w1 * w1 - n)
        d2 = 1.0 + f2 * (w2 * w2 - n)
        vals = (d0, b01 - a2, b02 + a1, t0,
                b01 + a2, d1, b12 - a0, t1,
                b02 - a1, b12 + a0, d2, t2)
        base = (g >> 3) * 512 + (g & 7) * 16
        for k, v in enumerate(vals):
            r, c2 = k // 4, k % 4
            out_v[r, pl.ds(c2 * 128 + base, 16)] = v
        return carry

    lax.fori_loop(0, _GROUPS, group, 0)
    pltpu.sync_copy(out_v, out_hbm.at[:, pl.ds(wid * 16 * _CHUNK, 16 * _CHUNK)])


_sc_kernel = pl.kernel(
    _sc_body,
    out_type=jax.ShapeDtypeStruct((3, _BATCH * 4), jnp.float32),
    mesh=plsc.VectorSubcoreMesh(core_axis_name="c", subcore_axis_name="s"),
    compiler_params=pltpu.CompilerParams(
        needs_layout_passes=False, use_tc_tiling_on_sc=False),
    scratch_types=[
        pltpu.VMEM((_CHUNKS, _CHUNK), jnp.int32),
        pltpu.VMEM((_CHUNKS, _CHUNK), jnp.int32),
        pltpu.VMEM((6 * _CHUNKS * _CHUNK, _TW), jnp.float32),
        pltpu.VMEM((3, 16 * _CHUNK), jnp.float32),
        pltpu.SemaphoreType.DMA,
    ],
)


@jax.jit
def kernel(indices, pose_adjustment):
    idx = indices.astype(jnp.int32).reshape(_NW, _CHUNKS, _CHUNK)
    traw = jnp.pad(pose_adjustment.T, ((0, 2), (0, 96)))
    table = traw.reshape(_TW, _TILES, _CHUNK).transpose(1, 0, 2)
    table = table.reshape(_TILES * _CHUNK, _TW)
    out = _sc_kernel(idx, table)
    out4 = out.reshape(3, _BATCH // _CHUNK, 4, _CHUNK)
    return out4.transpose(1, 3, 0, 2).reshape(_BATCH, 3, 4)
